# baseline (speedup 1.0000x reference)
import functools

import jax
import jax.numpy as jnp
from jax import lax
from jax.experimental import pallas as pl
from jax.experimental.pallas import tpu as pltpu
from jax.experimental.pallas import tpu_sc as plsc

BATCH = 16384
EMBED = 64
NUSERS = 100000
LANES = 16
NUM_WORKERS = 32
NCHUNK = BATCH // LANES

SEG_STARTS = (0, 40448, 80896)
SEG_LENS = (40448, 40448, 19104)
SEG_BUF = 40448


def _mf_body(uid_hbm, vid_hbm, utT_hbm, vtT_hbm, part_hbm,
             idbuf, uvals, acc, buf_a, buf_b, sem_a, sem_b):
    wid = lax.axis_index("s") * 2 + lax.axis_index("c")
    iota = lax.iota(jnp.int32, LANES)
    zeros = jnp.zeros((LANES,), jnp.float32)

    def zero_acc(c, carry):
        acc[pl.ds(c * LANES, LANES)] = zeros
        return carry

    lax.fori_loop(0, NCHUNK, zero_acc, 0)

    bufs = (buf_a, buf_b)
    sems = (sem_a, sem_b)

    for j2 in range(2):
        j = wid * 2 + j2
        for phase in range(2):
            tab = utT_hbm if phase == 0 else vtT_hbm
            ids = uid_hbm if phase == 0 else vid_hbm
            pltpu.sync_copy(ids, idbuf)
            copies = [None, None, None]
            copies[0] = pltpu.async_copy(
                tab.at[j, pl.ds(SEG_STARTS[0], SEG_LENS[0])],
                bufs[0].at[pl.ds(0, SEG_LENS[0])], sems[0])
            for s in range(3):
                copies[s].wait()
                if s < 2:
                    copies[s + 1] = pltpu.async_copy(
                        tab.at[j, pl.ds(SEG_STARTS[s + 1], SEG_LENS[s + 1])],
                        bufs[(s + 1) % 2].at[pl.ds(0, SEG_LENS[s + 1])],
                        sems[(s + 1) % 2])
                buf = bufs[s % 2]
                s0 = SEG_STARTS[s]
                slen = SEG_LENS[s]

                if phase == 0:
                    def chunk(c, carry, buf=buf, s0=s0, slen=slen):
                        loc = idbuf[pl.ds(c * LANES, LANES)] - s0
                        m = (loc >= 0) & (loc < slen)
                        g = plsc.load_gather(
                            buf, [jnp.where(m, loc, 0)], mask=m)
                        plsc.store_scatter(
                            uvals, [iota + c * LANES], g, mask=m)
                        return carry
                else:
                    def chunk(c, carry, buf=buf, s0=s0, slen=slen):
                        loc = idbuf[pl.ds(c * LANES, LANES)] - s0
                        m = (loc >= 0) & (loc < slen)
                        g = plsc.load_gather(
                            buf, [jnp.where(m, loc, 0)], mask=m)
                        uv = uvals[pl.ds(c * LANES, LANES)]
                        plsc.addupdate_scatter(
                            acc, [iota + c * LANES], g * uv, mask=m)
                        return carry

                lax.fori_loop(0, NCHUNK, chunk, 0)

    pltpu.sync_copy(acc, part_hbm.at[pl.ds(wid * BATCH, BATCH)])


def _reduce_body(part_hbm, out_hbm, rows_v, out_v, sem):
    wid = lax.axis_index("s") * 2 + lax.axis_index("c")
    n = BATCH // NUM_WORKERS
    base = wid * n
    pltpu.async_copy(part_hbm.at[:, pl.ds(base, n)], rows_v, sem).wait()

    def chunk(c, carry):
        a = jnp.zeros((LANES,), jnp.float32)
        for r in range(NUM_WORKERS):
            a = a + rows_v[r, pl.ds(c * LANES, LANES)]
        out_v[pl.ds(c * LANES, LANES)] = a
        return carry

    lax.fori_loop(0, n // LANES, chunk, 0)
    pltpu.sync_copy(out_v, out_hbm.at[pl.ds(base, n)])


@jax.jit
def kernel(user_id, video_id, user_table, video_table):
    uid = user_id.astype(jnp.int32)
    vid = video_id.astype(jnp.int32)
    utT = user_table.T
    vtT = video_table.T
    mesh = plsc.VectorSubcoreMesh(core_axis_name="c", subcore_axis_name="s")
    params = pltpu.CompilerParams(
        needs_layout_passes=False, use_tc_tiling_on_sc=False)

    mf = functools.partial(
        pl.kernel,
        mesh=mesh,
        compiler_params=params,
        out_type=jax.ShapeDtypeStruct((NUM_WORKERS * BATCH,), jnp.float32),
        scratch_types=[
            pltpu.VMEM((BATCH,), jnp.int32),
            pltpu.VMEM((BATCH,), jnp.float32),
            pltpu.VMEM((BATCH,), jnp.float32),
            pltpu.VMEM((SEG_BUF,), jnp.float32),
            pltpu.VMEM((SEG_BUF,), jnp.float32),
            pltpu.SemaphoreType.DMA,
            pltpu.SemaphoreType.DMA,
        ],
    )(_mf_body)
    partials = mf(uid, vid, utT, vtT).reshape(NUM_WORKERS, BATCH)

    red = functools.partial(
        pl.kernel,
        mesh=mesh,
        compiler_params=params,
        out_type=jax.ShapeDtypeStruct((BATCH,), jnp.float32),
        scratch_types=[
            pltpu.VMEM((NUM_WORKERS, BATCH // NUM_WORKERS), jnp.float32),
            pltpu.VMEM((BATCH // NUM_WORKERS,), jnp.float32),
            pltpu.SemaphoreType.DMA,
        ],
    )(_reduce_body)
    return red(partials)


# speedup vs baseline: 1.0514x; 1.0514x over previous
import functools

import jax
import jax.numpy as jnp
from jax import lax
from jax.experimental import pallas as pl
from jax.experimental.pallas import tpu as pltpu
from jax.experimental.pallas import tpu_sc as plsc

BATCH = 16384
EMBED = 64
NUSERS = 100000
LANES = 16
NUM_WORKERS = 32
NCHUNK = BATCH // LANES
UNROLL = 8

SEG_STARTS = (0, 40448, 80896)
SEG_LENS = (40448, 40448, 19104)
SEG_BUF = 40448


def _mf_body(uid_hbm, vid_hbm, utT_hbm, vtT_hbm, part_hbm,
             idbuf, uvals, acc, buf_a, buf_b, sem_a, sem_b):
    wid = lax.axis_index("s") * 2 + lax.axis_index("c")
    iota = lax.iota(jnp.int32, LANES)
    zeros = jnp.zeros((LANES,), jnp.float32)

    def zero_acc(c8, carry):
        for k in range(UNROLL):
            acc[pl.ds((c8 * UNROLL + k) * LANES, LANES)] = zeros
        return carry

    lax.fori_loop(0, NCHUNK // UNROLL, zero_acc, 0)

    bufs = (buf_a, buf_b)
    sems = (sem_a, sem_b)

    for j2 in range(2):
        j = wid * 2 + j2
        for phase in range(2):
            tab = utT_hbm if phase == 0 else vtT_hbm
            ids = uid_hbm if phase == 0 else vid_hbm
            pltpu.sync_copy(ids, idbuf)
            copies = [None, None, None]
            copies[0] = pltpu.async_copy(
                tab.at[j, pl.ds(SEG_STARTS[0], SEG_LENS[0])],
                bufs[0].at[pl.ds(0, SEG_LENS[0])], sems[0])
            for s in range(3):
                copies[s].wait()
                if s < 2:
                    copies[s + 1] = pltpu.async_copy(
                        tab.at[j, pl.ds(SEG_STARTS[s + 1], SEG_LENS[s + 1])],
                        bufs[(s + 1) % 2].at[pl.ds(0, SEG_LENS[s + 1])],
                        sems[(s + 1) % 2])
                buf = bufs[s % 2]
                s0 = SEG_STARTS[s]
                slen = SEG_LENS[s]

                if phase == 0:
                    def chunk(c8, carry, buf=buf, s0=s0, slen=slen):
                        for k in range(UNROLL):
                            c = c8 * UNROLL + k
                            loc = idbuf[pl.ds(c * LANES, LANES)] - s0
                            m = (loc >= 0) & (loc < slen)
                            g = plsc.load_gather(
                                buf, [jnp.where(m, loc, 0)], mask=m)
                            plsc.store_scatter(
                                uvals, [iota + c * LANES], g, mask=m)
                        return carry
                else:
                    def chunk(c8, carry, buf=buf, s0=s0, slen=slen):
                        for k in range(UNROLL):
                            c = c8 * UNROLL + k
                            loc = idbuf[pl.ds(c * LANES, LANES)] - s0
                            m = (loc >= 0) & (loc < slen)
                            g = plsc.load_gather(
                                buf, [jnp.where(m, loc, 0)], mask=m)
                            uv = uvals[pl.ds(c * LANES, LANES)]
                            plsc.addupdate_scatter(
                                acc, [iota + c * LANES], g * uv, mask=m)
                        return carry

                lax.fori_loop(0, NCHUNK // UNROLL, chunk, 0)

    pltpu.sync_copy(acc, part_hbm.at[pl.ds(wid * BATCH, BATCH)])


def _reduce_body(part_hbm, out_hbm, rows_v, out_v, sem):
    wid = lax.axis_index("s") * 2 + lax.axis_index("c")
    n = BATCH // NUM_WORKERS
    base = wid * n
    pltpu.async_copy(part_hbm.at[:, pl.ds(base, n)], rows_v, sem).wait()

    def chunk(c, carry):
        a = jnp.zeros((LANES,), jnp.float32)
        for r in range(NUM_WORKERS):
            a = a + rows_v[r, pl.ds(c * LANES, LANES)]
        out_v[pl.ds(c * LANES, LANES)] = a
        return carry

    lax.fori_loop(0, n // LANES, chunk, 0)
    pltpu.sync_copy(out_v, out_hbm.at[pl.ds(base, n)])


@jax.jit
def kernel(user_id, video_id, user_table, video_table):
    uid = user_id.astype(jnp.int32)
    vid = video_id.astype(jnp.int32)
    utT = user_table.T
    vtT = video_table.T
    mesh = plsc.VectorSubcoreMesh(core_axis_name="c", subcore_axis_name="s")
    params = pltpu.CompilerParams(
        needs_layout_passes=False, use_tc_tiling_on_sc=False)

    mf = functools.partial(
        pl.kernel,
        mesh=mesh,
        compiler_params=params,
        out_type=jax.ShapeDtypeStruct((NUM_WORKERS * BATCH,), jnp.float32),
        scratch_types=[
            pltpu.VMEM((BATCH,), jnp.int32),
            pltpu.VMEM((BATCH,), jnp.float32),
            pltpu.VMEM((BATCH,), jnp.float32),
            pltpu.VMEM((SEG_BUF,), jnp.float32),
            pltpu.VMEM((SEG_BUF,), jnp.float32),
            pltpu.SemaphoreType.DMA,
            pltpu.SemaphoreType.DMA,
        ],
    )(_mf_body)
    partials = mf(uid, vid, utT, vtT).reshape(NUM_WORKERS, BATCH)

    red = functools.partial(
        pl.kernel,
        mesh=mesh,
        compiler_params=params,
        out_type=jax.ShapeDtypeStruct((BATCH,), jnp.float32),
        scratch_types=[
            pltpu.VMEM((NUM_WORKERS, BATCH // NUM_WORKERS), jnp.float32),
            pltpu.VMEM((BATCH // NUM_WORKERS,), jnp.float32),
            pltpu.SemaphoreType.DMA,
        ],
    )(_reduce_body)
    return red(partials)


# speedup vs baseline: 1.3931x; 1.3250x over previous
import functools

import jax
import jax.numpy as jnp
from jax import lax
from jax.experimental import pallas as pl
from jax.experimental.pallas import tpu as pltpu
from jax.experimental.pallas import tpu_sc as plsc

BATCH = 16384
EMBED = 64
NUSERS = 100000
LANES = 16
NUM_WORKERS = 32
NCHUNK = BATCH // LANES
UNROLL = 8

SEG_STARTS = (0, 40448, 80896)
SEG_LENS = (40448, 40448, 19104)
SEG_BUF = 40448


def _mf_body(uid_hbm, vid_hbm, utT_hbm, vtT_hbm, part_hbm,
             idbuf, uvals, acc, buf_a, buf_b, sem_a, sem_b):
    wid = lax.axis_index("s") * 2 + lax.axis_index("c")
    iota = lax.iota(jnp.int32, LANES)
    zeros = jnp.zeros((LANES,), jnp.float32)

    def zero_acc(c8, carry):
        for k in range(UNROLL):
            acc[pl.ds((c8 * UNROLL + k) * LANES, LANES)] = zeros
        return carry

    lax.fori_loop(0, NCHUNK // UNROLL, zero_acc, 0)

    bufs = (buf_a, buf_b)
    sems = (sem_a, sem_b)

    for j2 in range(2):
        j = wid * 2 + j2
        for phase in range(2):
            tab = utT_hbm if phase == 0 else vtT_hbm
            ids = uid_hbm if phase == 0 else vid_hbm
            pltpu.sync_copy(ids, idbuf)
            copies = [None, None, None]
            copies[0] = pltpu.async_copy(
                tab.at[j, pl.ds(SEG_STARTS[0], SEG_LENS[0])],
                bufs[0].at[pl.ds(0, SEG_LENS[0])], sems[0])
            for s in range(3):
                copies[s].wait()
                if s < 2:
                    copies[s + 1] = pltpu.async_copy(
                        tab.at[j, pl.ds(SEG_STARTS[s + 1], SEG_LENS[s + 1])],
                        bufs[(s + 1) % 2].at[pl.ds(0, SEG_LENS[s + 1])],
                        sems[(s + 1) % 2])
                buf = bufs[s % 2]
                s0 = SEG_STARTS[s]
                slen = SEG_LENS[s]

                if phase == 0:
                    def chunk(c8, carry, buf=buf, s0=s0, slen=slen):
                        for k in range(UNROLL):
                            c = c8 * UNROLL + k
                            loc = idbuf[pl.ds(c * LANES, LANES)] - s0
                            m = (loc >= 0) & (loc < slen)
                            g = plsc.load_gather(
                                buf, [jnp.where(m, loc, 0)], mask=m)
                            uv = uvals[pl.ds(c * LANES, LANES)]
                            uvals[pl.ds(c * LANES, LANES)] = (
                                jnp.where(m, g, uv))
                        return carry
                else:
                    def chunk(c8, carry, buf=buf, s0=s0, slen=slen):
                        for k in range(UNROLL):
                            c = c8 * UNROLL + k
                            loc = idbuf[pl.ds(c * LANES, LANES)] - s0
                            m = (loc >= 0) & (loc < slen)
                            g = plsc.load_gather(
                                buf, [jnp.where(m, loc, 0)], mask=m)
                            uv = uvals[pl.ds(c * LANES, LANES)]
                            a = acc[pl.ds(c * LANES, LANES)]
                            acc[pl.ds(c * LANES, LANES)] = (
                                a + jnp.where(m, g * uv, 0.0))
                        return carry

                lax.fori_loop(0, NCHUNK // UNROLL, chunk, 0)

    pltpu.sync_copy(acc, part_hbm.at[pl.ds(wid * BATCH, BATCH)])


def _reduce_body(part_hbm, out_hbm, rows_v, out_v, sem):
    wid = lax.axis_index("s") * 2 + lax.axis_index("c")
    n = BATCH // NUM_WORKERS
    base = wid * n
    pltpu.async_copy(part_hbm.at[:, pl.ds(base, n)], rows_v, sem).wait()

    def chunk(c, carry):
        a = jnp.zeros((LANES,), jnp.float32)
        for r in range(NUM_WORKERS):
            a = a + rows_v[r, pl.ds(c * LANES, LANES)]
        out_v[pl.ds(c * LANES, LANES)] = a
        return carry

    lax.fori_loop(0, n // LANES, chunk, 0)
    pltpu.sync_copy(out_v, out_hbm.at[pl.ds(base, n)])


@jax.jit
def kernel(user_id, video_id, user_table, video_table):
    uid = user_id.astype(jnp.int32)
    vid = video_id.astype(jnp.int32)
    utT = user_table.T
    vtT = video_table.T
    mesh = plsc.VectorSubcoreMesh(core_axis_name="c", subcore_axis_name="s")
    params = pltpu.CompilerParams(
        needs_layout_passes=False, use_tc_tiling_on_sc=False)

    mf = functools.partial(
        pl.kernel,
        mesh=mesh,
        compiler_params=params,
        out_type=jax.ShapeDtypeStruct((NUM_WORKERS * BATCH,), jnp.float32),
        scratch_types=[
            pltpu.VMEM((BATCH,), jnp.int32),
            pltpu.VMEM((BATCH,), jnp.float32),
            pltpu.VMEM((BATCH,), jnp.float32),
            pltpu.VMEM((SEG_BUF,), jnp.float32),
            pltpu.VMEM((SEG_BUF,), jnp.float32),
            pltpu.SemaphoreType.DMA,
            pltpu.SemaphoreType.DMA,
        ],
    )(_mf_body)
    partials = mf(uid, vid, utT, vtT).reshape(NUM_WORKERS, BATCH)

    red = functools.partial(
        pl.kernel,
        mesh=mesh,
        compiler_params=params,
        out_type=jax.ShapeDtypeStruct((BATCH,), jnp.float32),
        scratch_types=[
            pltpu.VMEM((NUM_WORKERS, BATCH // NUM_WORKERS), jnp.float32),
            pltpu.VMEM((BATCH // NUM_WORKERS,), jnp.float32),
            pltpu.SemaphoreType.DMA,
        ],
    )(_reduce_body)
    return red(partials)
